# baseline (speedup 1.0000x reference)
import functools

import jax
import jax.numpy as jnp
from jax import lax
from jax.experimental import pallas as pl
from jax.experimental.pallas import tpu as pltpu
from jax.experimental.pallas import tpu_sc as plsc

_D = 128
_S_SELF = 4
_S_NEIGH = 32
_N = 10000
_NSC = 3200
_NTC = _N - _NSC
_BLK = 400
_BLK2 = 400
_NW = 32
_C = 4
_PER_W = _NSC // _NW


def _tc1_body(s_ref, n_ref, w_ref, b_ref, o_ref):
    ssum = jnp.sum(s_ref[...], axis=1)
    nsum = jnp.sum(n_ref[...], axis=1)
    x = jnp.concatenate([ssum, nsum], axis=-1)
    y = jnp.dot(x, w_ref[...], preferred_element_type=jnp.float32)
    o_ref[...] = jnp.maximum(y + b_ref[...], 0.0)


def _tc2_body(sum_ref, w_ref, b_ref, prev_ref, o_ref):
    y = jnp.dot(sum_ref[...], w_ref[...], preferred_element_type=jnp.float32)
    o_ref[...] = jnp.maximum(y + b_ref[...], 0.0)


def _sc_body(self_hbm, neigh_hbm, out_hbm, nbuf, sbuf, obuf):
    wid = lax.axis_index("s") * 2 + lax.axis_index("c")
    node0 = _NTC + wid * _PER_W
    obase = wid * _PER_W

    def chunk(g, carry):
        src = node0 + g * _C
        dst = obase + g * _C
        pltpu.sync_copy(neigh_hbm.at[pl.ds(src, _C)], nbuf)
        pltpu.sync_copy(self_hbm.at[pl.ds(src, _C)], sbuf)
        for n in range(_C):
            for l in range(8):
                sl = pl.ds(l * 16, 16)
                sacc = sbuf[n, 0, sl]
                for r in range(1, _S_SELF):
                    sacc = sacc + sbuf[n, r, sl]
                obuf[n, sl] = sacc
                nacc = nbuf[n, 0, sl]
                for r in range(1, _S_NEIGH):
                    nacc = nacc + nbuf[n, r, sl]
                obuf[n, pl.ds(_D + l * 16, 16)] = nacc
        pltpu.sync_copy(obuf, out_hbm.at[pl.ds(dst, _C)])
        return carry

    lax.fori_loop(0, _PER_W // _C, chunk, 0)


_sc_sums = functools.partial(
    pl.kernel,
    out_type=jax.ShapeDtypeStruct((_NSC, 2 * _D), jnp.float32),
    mesh=plsc.VectorSubcoreMesh(core_axis_name="c", subcore_axis_name="s"),
    scratch_types=[
        pltpu.VMEM((_C, _S_NEIGH, _D), jnp.float32),
        pltpu.VMEM((_C, _S_SELF, _D), jnp.float32),
        pltpu.VMEM((_C, 2 * _D), jnp.float32),
    ],
)(_sc_body)


def kernel(self_vecs, neigh_vecs, W_neigh, b_neigh, W_self, b_self):
    n_nodes, s_self, d = self_vecs.shape
    s_neigh = neigh_vecs.shape[1]
    w = jnp.concatenate([W_self / s_self, W_neigh / s_neigh], axis=0)
    b = (b_self + b_neigh).reshape(1, d)

    sc_sums = _sc_sums(self_vecs, neigh_vecs)

    out_head = pl.pallas_call(
        _tc1_body,
        grid=(_NTC // _BLK,),
        in_specs=[
            pl.BlockSpec((_BLK, s_self, d), lambda i: (i, 0, 0)),
            pl.BlockSpec((_BLK, s_neigh, d), lambda i: (i, 0, 0)),
            pl.BlockSpec((2 * d, d), lambda i: (0, 0)),
            pl.BlockSpec((1, d), lambda i: (0, 0)),
        ],
        out_specs=pl.BlockSpec((_BLK, d), lambda i: (i, 0)),
        out_shape=jax.ShapeDtypeStruct((n_nodes, d), jnp.float32),
    )(self_vecs, neigh_vecs, w, b)

    return pl.pallas_call(
        _tc2_body,
        grid=(_NSC // _BLK2,),
        in_specs=[
            pl.BlockSpec((_BLK2, 2 * d), lambda i: (i, 0)),
            pl.BlockSpec((2 * d, d), lambda i: (0, 0)),
            pl.BlockSpec((1, d), lambda i: (0, 0)),
            pl.BlockSpec(memory_space=pl.ANY),
        ],
        out_specs=pl.BlockSpec((_BLK2, d), lambda i: (i + _NTC // _BLK2, 0)),
        out_shape=jax.ShapeDtypeStruct((n_nodes, d), jnp.float32),
        input_output_aliases={3: 0},
    )(sc_sums, w, b, out_head)


# speedup vs baseline: 1.3407x; 1.3407x over previous
import functools

import jax
import jax.numpy as jnp
from jax import lax
from jax.experimental import pallas as pl
from jax.experimental.pallas import tpu as pltpu
from jax.experimental.pallas import tpu_sc as plsc

_D = 128
_S_SELF = 4
_S_NEIGH = 32
_N = 10000
_NSC = 3200
_NTC = _N - _NSC
_BLK = 400
_BLK2 = 400
_NW = 32
_C = 4
_PER_W = _NSC // _NW


def _tc1_body(s_ref, n_ref, w_ref, b_ref, o_ref):
    ssum = jnp.sum(s_ref[...], axis=1)
    nsum = jnp.sum(n_ref[...], axis=1)
    x = jnp.concatenate([ssum, nsum], axis=-1)
    y = jnp.dot(x, w_ref[...], preferred_element_type=jnp.float32)
    o_ref[...] = jnp.maximum(y + b_ref[...], 0.0)


def _tc2_body(sum_ref, w_ref, b_ref, prev_ref, o_ref):
    y = jnp.dot(sum_ref[...], w_ref[...], preferred_element_type=jnp.float32)
    o_ref[...] = jnp.maximum(y + b_ref[...], 0.0)


def _sc_body(self_hbm, neigh_hbm, out_hbm,
             nbuf0, nbuf1, sbuf0, sbuf1, obuf,
             nsem0, nsem1, ssem0, ssem1):
    wid = lax.axis_index("s") * 2 + lax.axis_index("c")
    node0 = _NTC + wid * _PER_W
    obase = wid * _PER_W
    nchunks = _PER_W // _C
    nbufs, sbufs = (nbuf0, nbuf1), (sbuf0, sbuf1)
    nsems, ssems = (nsem0, nsem1), (ssem0, ssem1)

    def issue(g, b):
        src = node0 + g * _C
        pltpu.async_copy(neigh_hbm.at[pl.ds(src, _C)], nbufs[b], nsems[b])
        pltpu.async_copy(self_hbm.at[pl.ds(src, _C)], sbufs[b], ssems[b])

    issue(0, 0)
    issue(1, 1)

    def do_chunk(g, b, more):
        pltpu.make_async_copy(
            neigh_hbm.at[pl.ds(0, _C)], nbufs[b], nsems[b]).wait()
        pltpu.make_async_copy(
            self_hbm.at[pl.ds(0, _C)], sbufs[b], ssems[b]).wait()
        for n in range(_C):
            for l in range(8):
                sl = pl.ds(l * 16, 16)
                sacc = sbufs[b][n, 0, sl]
                for r in range(1, _S_SELF):
                    sacc = sacc + sbufs[b][n, r, sl]
                obuf[n, sl] = sacc
                nacc = nbufs[b][n, 0, sl]
                for r in range(1, _S_NEIGH):
                    nacc = nacc + nbufs[b][n, r, sl]
                obuf[n, pl.ds(_D + l * 16, 16)] = nacc

        if more:
            @pl.when(g + 2 < nchunks)
            def _():
                issue(g + 2, b)

        pltpu.sync_copy(obuf, out_hbm.at[pl.ds(obase + g * _C, _C)])

    def pair(p, carry):
        do_chunk(2 * p, 0, True)
        do_chunk(2 * p + 1, 1, True)
        return carry

    lax.fori_loop(0, nchunks // 2, pair, 0)
    if nchunks % 2:
        do_chunk(nchunks - 1, 0, False)


_sc_sums = functools.partial(
    pl.kernel,
    out_type=jax.ShapeDtypeStruct((_NSC, 2 * _D), jnp.float32),
    mesh=plsc.VectorSubcoreMesh(core_axis_name="c", subcore_axis_name="s"),
    scratch_types=[
        pltpu.VMEM((_C, _S_NEIGH, _D), jnp.float32),
        pltpu.VMEM((_C, _S_NEIGH, _D), jnp.float32),
        pltpu.VMEM((_C, _S_SELF, _D), jnp.float32),
        pltpu.VMEM((_C, _S_SELF, _D), jnp.float32),
        pltpu.VMEM((_C, 2 * _D), jnp.float32),
        pltpu.SemaphoreType.DMA,
        pltpu.SemaphoreType.DMA,
        pltpu.SemaphoreType.DMA,
        pltpu.SemaphoreType.DMA,
    ],
)(_sc_body)


def kernel(self_vecs, neigh_vecs, W_neigh, b_neigh, W_self, b_self):
    n_nodes, s_self, d = self_vecs.shape
    s_neigh = neigh_vecs.shape[1]
    w = jnp.concatenate([W_self / s_self, W_neigh / s_neigh], axis=0)
    b = (b_self + b_neigh).reshape(1, d)

    sc_sums = _sc_sums(self_vecs, neigh_vecs)

    out_head = pl.pallas_call(
        _tc1_body,
        grid=(_NTC // _BLK,),
        in_specs=[
            pl.BlockSpec((_BLK, s_self, d), lambda i: (i, 0, 0)),
            pl.BlockSpec((_BLK, s_neigh, d), lambda i: (i, 0, 0)),
            pl.BlockSpec((2 * d, d), lambda i: (0, 0)),
            pl.BlockSpec((1, d), lambda i: (0, 0)),
        ],
        out_specs=pl.BlockSpec((_BLK, d), lambda i: (i, 0)),
        out_shape=jax.ShapeDtypeStruct((n_nodes, d), jnp.float32),
    )(self_vecs, neigh_vecs, w, b)

    return pl.pallas_call(
        _tc2_body,
        grid=(_NSC // _BLK2,),
        in_specs=[
            pl.BlockSpec((_BLK2, 2 * d), lambda i: (i, 0)),
            pl.BlockSpec((2 * d, d), lambda i: (0, 0)),
            pl.BlockSpec((1, d), lambda i: (0, 0)),
            pl.BlockSpec(memory_space=pl.ANY),
        ],
        out_specs=pl.BlockSpec((_BLK2, d), lambda i: (i + _NTC // _BLK2, 0)),
        out_shape=jax.ShapeDtypeStruct((n_nodes, d), jnp.float32),
        input_output_aliases={3: 0},
    )(sc_sums, w, b, out_head)


# speedup vs baseline: 1.3489x; 1.0061x over previous
import functools

import jax
import jax.numpy as jnp
from jax import lax
from jax.experimental import pallas as pl
from jax.experimental.pallas import tpu as pltpu
from jax.experimental.pallas import tpu_sc as plsc

_D = 128
_S_SELF = 4
_S_NEIGH = 32
_N = 10000
_NSC = 3200
_NTC = _N - _NSC
_BLK = 400
_BLK2 = 400
_NW = 32
_C = 4
_PER_W = _NSC // _NW


def _tc1_body(s_ref, n_ref, w_ref, b_ref, o_ref):
    ssum = jnp.sum(s_ref[...], axis=1)
    nsum = jnp.sum(n_ref[...], axis=1)
    x = jnp.concatenate([ssum, nsum], axis=-1)
    y = jnp.dot(x, w_ref[...], preferred_element_type=jnp.float32)
    o_ref[...] = jnp.maximum(y + b_ref[...], 0.0)


def _tc2_body(sum_ref, w_ref, b_ref, prev_ref, o_ref):
    y = jnp.dot(sum_ref[...], w_ref[...], preferred_element_type=jnp.float32)
    o_ref[...] = jnp.maximum(y + b_ref[...], 0.0)


def _sc_body(self_hbm, neigh_hbm, out_hbm,
             nbuf0, nbuf1, sbuf0, sbuf1, obuf0, obuf1,
             nsem0, nsem1, ssem0, ssem1, osem0, osem1):
    wid = lax.axis_index("s") * 2 + lax.axis_index("c")
    node0 = _NTC + wid * _PER_W
    obase = wid * _PER_W
    nchunks = _PER_W // _C
    nbufs, sbufs = (nbuf0, nbuf1), (sbuf0, sbuf1)
    obufs = (obuf0, obuf1)
    nsems, ssems = (nsem0, nsem1), (ssem0, ssem1)
    osems = (osem0, osem1)

    def issue(g, b):
        src = node0 + g * _C
        pltpu.async_copy(neigh_hbm.at[pl.ds(src, _C)], nbufs[b], nsems[b])
        pltpu.async_copy(self_hbm.at[pl.ds(src, _C)], sbufs[b], ssems[b])

    issue(0, 0)
    issue(1, 1)

    def drain_out(b):
        pltpu.make_async_copy(
            obufs[b], out_hbm.at[pl.ds(0, _C)], osems[b]).wait()

    def do_chunk(g, b, more):
        pltpu.make_async_copy(
            neigh_hbm.at[pl.ds(0, _C)], nbufs[b], nsems[b]).wait()
        pltpu.make_async_copy(
            self_hbm.at[pl.ds(0, _C)], sbufs[b], ssems[b]).wait()

        if more:
            @pl.when(g >= 2)
            def _():
                drain_out(b)
        else:
            drain_out(b)

        for n in range(_C):
            for l in range(8):
                sl = pl.ds(l * 16, 16)
                sacc = sbufs[b][n, 0, sl]
                for r in range(1, _S_SELF):
                    sacc = sacc + sbufs[b][n, r, sl]
                obufs[b][n, sl] = sacc
                nacc = nbufs[b][n, 0, sl]
                for r in range(1, _S_NEIGH):
                    nacc = nacc + nbufs[b][n, r, sl]
                obufs[b][n, pl.ds(_D + l * 16, 16)] = nacc

        if more:
            @pl.when(g + 2 < nchunks)
            def _():
                issue(g + 2, b)

        pltpu.async_copy(
            obufs[b], out_hbm.at[pl.ds(obase + g * _C, _C)], osems[b])

    def pair(p, carry):
        do_chunk(2 * p, 0, True)
        do_chunk(2 * p + 1, 1, True)
        return carry

    lax.fori_loop(0, nchunks // 2, pair, 0)
    if nchunks % 2:
        do_chunk(nchunks - 1, 0, False)
    drain_out(0)
    drain_out(1)


_sc_sums = functools.partial(
    pl.kernel,
    out_type=jax.ShapeDtypeStruct((_NSC, 2 * _D), jnp.float32),
    mesh=plsc.VectorSubcoreMesh(core_axis_name="c", subcore_axis_name="s"),
    scratch_types=[
        pltpu.VMEM((_C, _S_NEIGH, _D), jnp.float32),
        pltpu.VMEM((_C, _S_NEIGH, _D), jnp.float32),
        pltpu.VMEM((_C, _S_SELF, _D), jnp.float32),
        pltpu.VMEM((_C, _S_SELF, _D), jnp.float32),
        pltpu.VMEM((_C, 2 * _D), jnp.float32),
        pltpu.VMEM((_C, 2 * _D), jnp.float32),
        pltpu.SemaphoreType.DMA,
        pltpu.SemaphoreType.DMA,
        pltpu.SemaphoreType.DMA,
        pltpu.SemaphoreType.DMA,
        pltpu.SemaphoreType.DMA,
        pltpu.SemaphoreType.DMA,
    ],
)(_sc_body)


def kernel(self_vecs, neigh_vecs, W_neigh, b_neigh, W_self, b_self):
    n_nodes, s_self, d = self_vecs.shape
    s_neigh = neigh_vecs.shape[1]
    w = jnp.concatenate([W_self / s_self, W_neigh / s_neigh], axis=0)
    b = (b_self + b_neigh).reshape(1, d)

    sc_sums = _sc_sums(self_vecs, neigh_vecs)

    out_head = pl.pallas_call(
        _tc1_body,
        grid=(_NTC // _BLK,),
        in_specs=[
            pl.BlockSpec((_BLK, s_self, d), lambda i: (i, 0, 0)),
            pl.BlockSpec((_BLK, s_neigh, d), lambda i: (i, 0, 0)),
            pl.BlockSpec((2 * d, d), lambda i: (0, 0)),
            pl.BlockSpec((1, d), lambda i: (0, 0)),
        ],
        out_specs=pl.BlockSpec((_BLK, d), lambda i: (i, 0)),
        out_shape=jax.ShapeDtypeStruct((n_nodes, d), jnp.float32),
    )(self_vecs, neigh_vecs, w, b)

    return pl.pallas_call(
        _tc2_body,
        grid=(_NSC // _BLK2,),
        in_specs=[
            pl.BlockSpec((_BLK2, 2 * d), lambda i: (i, 0)),
            pl.BlockSpec((2 * d, d), lambda i: (0, 0)),
            pl.BlockSpec((1, d), lambda i: (0, 0)),
            pl.BlockSpec(memory_space=pl.ANY),
        ],
        out_specs=pl.BlockSpec((_BLK2, d), lambda i: (i + _NTC // _BLK2, 0)),
        out_shape=jax.ShapeDtypeStruct((n_nodes, d), jnp.float32),
        input_output_aliases={3: 0},
    )(sc_sums, w, b, out_head)


# speedup vs baseline: 1.5402x; 1.1419x over previous
import functools

import jax
import jax.numpy as jnp
from jax import lax
from jax.experimental import pallas as pl
from jax.experimental.pallas import tpu as pltpu
from jax.experimental.pallas import tpu_sc as plsc

_D = 128
_S_SELF = 4
_S_NEIGH = 32
_N = 10000
_NSC = 2560
_NTC = _N - _NSC
_BLK = 240
_BLK2 = 80
_NW = 32
_C = 8
_PER_W = _NSC // _NW


def _tc1_body(s_ref, n_ref, w_ref, b_ref, o_ref):
    ssum = jnp.sum(s_ref[...], axis=1)
    nsum = jnp.sum(n_ref[...], axis=1)
    x = jnp.concatenate([ssum, nsum], axis=-1)
    y = jnp.dot(x, w_ref[...], preferred_element_type=jnp.float32)
    o_ref[...] = jnp.maximum(y + b_ref[...], 0.0)


def _tc2_body(sum_ref, w_ref, b_ref, prev_ref, o_ref):
    y = jnp.dot(sum_ref[...], w_ref[...], preferred_element_type=jnp.float32)
    o_ref[...] = jnp.maximum(y + b_ref[...], 0.0)


def _sc_body(self_hbm, neigh_hbm, out_hbm,
             nbuf0, nbuf1, sbuf0, sbuf1, obuf0, obuf1,
             nsem0, nsem1, ssem0, ssem1, osem0, osem1):
    wid = lax.axis_index("s") * 2 + lax.axis_index("c")
    node0 = _NTC + wid * _PER_W
    obase = wid * _PER_W
    nchunks = _PER_W // _C
    nbufs, sbufs = (nbuf0, nbuf1), (sbuf0, sbuf1)
    obufs = (obuf0, obuf1)
    nsems, ssems = (nsem0, nsem1), (ssem0, ssem1)
    osems = (osem0, osem1)

    def issue(g, b):
        src = node0 + g * _C
        pltpu.async_copy(neigh_hbm.at[pl.ds(src, _C)], nbufs[b], nsems[b])
        pltpu.async_copy(self_hbm.at[pl.ds(src, _C)], sbufs[b], ssems[b])

    issue(0, 0)
    issue(1, 1)

    def drain_out(b):
        pltpu.make_async_copy(
            obufs[b], out_hbm.at[pl.ds(0, _C)], osems[b]).wait()

    def do_chunk(g, b, more):
        pltpu.make_async_copy(
            neigh_hbm.at[pl.ds(0, _C)], nbufs[b], nsems[b]).wait()
        pltpu.make_async_copy(
            self_hbm.at[pl.ds(0, _C)], sbufs[b], ssems[b]).wait()

        if more:
            @pl.when(g >= 2)
            def _():
                drain_out(b)
        else:
            drain_out(b)

        def node_body(n, carry):
            for l in range(8):
                sl = pl.ds(l * 16, 16)
                s0 = sbufs[b][n, 0, sl] + sbufs[b][n, 1, sl]
                s1 = sbufs[b][n, 2, sl] + sbufs[b][n, 3, sl]
                obufs[b][n, sl] = s0 + s1
                parts = []
                for k in range(4):
                    acc = nbufs[b][n, 8 * k, sl]
                    for r in range(8 * k + 1, 8 * k + 8):
                        acc = acc + nbufs[b][n, r, sl]
                    parts.append(acc)
                obufs[b][n, pl.ds(_D + l * 16, 16)] = (
                    (parts[0] + parts[1]) + (parts[2] + parts[3]))
            return carry

        lax.fori_loop(0, _C, node_body, 0)

        if more:
            @pl.when(g + 2 < nchunks)
            def _():
                issue(g + 2, b)

        pltpu.async_copy(
            obufs[b], out_hbm.at[pl.ds(obase + g * _C, _C)], osems[b])

    def pair(p, carry):
        do_chunk(2 * p, 0, True)
        do_chunk(2 * p + 1, 1, True)
        return carry

    lax.fori_loop(0, nchunks // 2, pair, 0)
    if nchunks % 2:
        do_chunk(nchunks - 1, 0, False)
    drain_out(0)
    drain_out(1)


_sc_sums = functools.partial(
    pl.kernel,
    out_type=jax.ShapeDtypeStruct((_NSC, 2 * _D), jnp.float32),
    mesh=plsc.VectorSubcoreMesh(core_axis_name="c", subcore_axis_name="s"),
    scratch_types=[
        pltpu.VMEM((_C, _S_NEIGH, _D), jnp.float32),
        pltpu.VMEM((_C, _S_NEIGH, _D), jnp.float32),
        pltpu.VMEM((_C, _S_SELF, _D), jnp.float32),
        pltpu.VMEM((_C, _S_SELF, _D), jnp.float32),
        pltpu.VMEM((_C, 2 * _D), jnp.float32),
        pltpu.VMEM((_C, 2 * _D), jnp.float32),
        pltpu.SemaphoreType.DMA,
        pltpu.SemaphoreType.DMA,
        pltpu.SemaphoreType.DMA,
        pltpu.SemaphoreType.DMA,
        pltpu.SemaphoreType.DMA,
        pltpu.SemaphoreType.DMA,
    ],
)(_sc_body)


def kernel(self_vecs, neigh_vecs, W_neigh, b_neigh, W_self, b_self):
    n_nodes, s_self, d = self_vecs.shape
    s_neigh = neigh_vecs.shape[1]
    w = jnp.concatenate([W_self / s_self, W_neigh / s_neigh], axis=0)
    b = (b_self + b_neigh).reshape(1, d)

    sc_sums = _sc_sums(self_vecs, neigh_vecs)

    out_head = pl.pallas_call(
        _tc1_body,
        grid=(_NTC // _BLK,),
        in_specs=[
            pl.BlockSpec((_BLK, s_self, d), lambda i: (i, 0, 0)),
            pl.BlockSpec((_BLK, s_neigh, d), lambda i: (i, 0, 0)),
            pl.BlockSpec((2 * d, d), lambda i: (0, 0)),
            pl.BlockSpec((1, d), lambda i: (0, 0)),
        ],
        out_specs=pl.BlockSpec((_BLK, d), lambda i: (i, 0)),
        out_shape=jax.ShapeDtypeStruct((n_nodes, d), jnp.float32),
    )(self_vecs, neigh_vecs, w, b)

    return pl.pallas_call(
        _tc2_body,
        grid=(_NSC // _BLK2,),
        in_specs=[
            pl.BlockSpec((_BLK2, 2 * d), lambda i: (i, 0)),
            pl.BlockSpec((2 * d, d), lambda i: (0, 0)),
            pl.BlockSpec((1, d), lambda i: (0, 0)),
            pl.BlockSpec(memory_space=pl.ANY),
        ],
        out_specs=pl.BlockSpec((_BLK2, d), lambda i: (i + _NTC // _BLK2, 0)),
        out_shape=jax.ShapeDtypeStruct((n_nodes, d), jnp.float32),
        input_output_aliases={3: 0},
    )(sc_sums, w, b, out_head)


# speedup vs baseline: 1.8367x; 1.1925x over previous
import functools

import jax
import jax.numpy as jnp
from jax import lax
from jax.experimental import pallas as pl
from jax.experimental.pallas import tpu as pltpu
from jax.experimental.pallas import tpu_sc as plsc

_D = 128
_S_SELF = 4
_S_NEIGH = 32
_N = 10000
_NSC = 3200
_NTC = _N - _NSC
_BLK = 400
_BLK2 = 400
_NW = 32
_C = 4
_PER_W = _NSC // _NW


def _tc1_body(s_ref, n_ref, w_ref, b_ref, o_ref):
    ssum = jnp.sum(s_ref[...], axis=1)
    nsum = jnp.sum(n_ref[...], axis=1)
    x = jnp.concatenate([ssum, nsum], axis=-1)
    y = jnp.dot(x, w_ref[...], preferred_element_type=jnp.float32)
    o_ref[...] = jnp.maximum(y + b_ref[...], 0.0)


def _tc2_body(sum_ref, w_ref, b_ref, prev_ref, o_ref):
    y = jnp.dot(sum_ref[...], w_ref[...], preferred_element_type=jnp.float32)
    o_ref[...] = jnp.maximum(y + b_ref[...], 0.0)


def _sc_body(self_hbm, neigh_hbm, out_hbm,
             nbuf0, nbuf1, sbuf0, sbuf1, obuf0, obuf1,
             nsem0, nsem1, ssem0, ssem1, osem0, osem1):
    wid = lax.axis_index("s") * 2 + lax.axis_index("c")
    node0 = _NTC + wid * _PER_W
    obase = wid * _PER_W
    nchunks = _PER_W // _C
    nbufs, sbufs = (nbuf0, nbuf1), (sbuf0, sbuf1)
    obufs = (obuf0, obuf1)
    nsems, ssems = (nsem0, nsem1), (ssem0, ssem1)
    osems = (osem0, osem1)

    def issue(g, b):
        src = node0 + g * _C
        pltpu.async_copy(neigh_hbm.at[pl.ds(src, _C)], nbufs[b], nsems[b])
        pltpu.async_copy(self_hbm.at[pl.ds(src, _C)], sbufs[b], ssems[b])

    issue(0, 0)
    issue(1, 1)

    def drain_out(b):
        pltpu.make_async_copy(
            obufs[b], out_hbm.at[pl.ds(0, _C)], osems[b]).wait()

    def do_chunk(g, b, more):
        pltpu.make_async_copy(
            neigh_hbm.at[pl.ds(0, _C)], nbufs[b], nsems[b]).wait()
        pltpu.make_async_copy(
            self_hbm.at[pl.ds(0, _C)], sbufs[b], ssems[b]).wait()

        if more:
            @pl.when(g >= 2)
            def _():
                drain_out(b)
        else:
            drain_out(b)

        def node_body(n, carry):
            for l in range(8):
                sl = pl.ds(l * 16, 16)
                s0 = sbufs[b][n, 0, sl] + sbufs[b][n, 1, sl]
                s1 = sbufs[b][n, 2, sl] + sbufs[b][n, 3, sl]
                obufs[b][n, sl] = s0 + s1
                parts = []
                for k in range(4):
                    acc = nbufs[b][n, 8 * k, sl]
                    for r in range(8 * k + 1, 8 * k + 8):
                        acc = acc + nbufs[b][n, r, sl]
                    parts.append(acc)
                obufs[b][n, pl.ds(_D + l * 16, 16)] = (
                    (parts[0] + parts[1]) + (parts[2] + parts[3]))
            return carry

        lax.fori_loop(0, _C, node_body, 0)

        if more:
            @pl.when(g + 2 < nchunks)
            def _():
                issue(g + 2, b)

        pltpu.async_copy(
            obufs[b], out_hbm.at[pl.ds(obase + g * _C, _C)], osems[b])

    def pair(p, carry):
        do_chunk(2 * p, 0, True)
        do_chunk(2 * p + 1, 1, True)
        return carry

    lax.fori_loop(0, nchunks // 2, pair, 0)
    if nchunks % 2:
        do_chunk(nchunks - 1, 0, False)
    drain_out(0)
    drain_out(1)


_sc_sums = functools.partial(
    pl.kernel,
    out_type=jax.ShapeDtypeStruct((_NSC, 2 * _D), jnp.float32),
    mesh=plsc.VectorSubcoreMesh(core_axis_name="c", subcore_axis_name="s"),
    scratch_types=[
        pltpu.VMEM((_C, _S_NEIGH, _D), jnp.float32),
        pltpu.VMEM((_C, _S_NEIGH, _D), jnp.float32),
        pltpu.VMEM((_C, _S_SELF, _D), jnp.float32),
        pltpu.VMEM((_C, _S_SELF, _D), jnp.float32),
        pltpu.VMEM((_C, 2 * _D), jnp.float32),
        pltpu.VMEM((_C, 2 * _D), jnp.float32),
        pltpu.SemaphoreType.DMA,
        pltpu.SemaphoreType.DMA,
        pltpu.SemaphoreType.DMA,
        pltpu.SemaphoreType.DMA,
        pltpu.SemaphoreType.DMA,
        pltpu.SemaphoreType.DMA,
    ],
)(_sc_body)


def kernel(self_vecs, neigh_vecs, W_neigh, b_neigh, W_self, b_self):
    n_nodes, s_self, d = self_vecs.shape
    s_neigh = neigh_vecs.shape[1]
    w = jnp.concatenate([W_self / s_self, W_neigh / s_neigh], axis=0)
    b = (b_self + b_neigh).reshape(1, d)

    sc_sums = _sc_sums(self_vecs, neigh_vecs)

    out_head = pl.pallas_call(
        _tc1_body,
        grid=(_NTC // _BLK,),
        in_specs=[
            pl.BlockSpec((_BLK, s_self, d), lambda i: (i, 0, 0)),
            pl.BlockSpec((_BLK, s_neigh, d), lambda i: (i, 0, 0)),
            pl.BlockSpec((2 * d, d), lambda i: (0, 0)),
            pl.BlockSpec((1, d), lambda i: (0, 0)),
        ],
        out_specs=pl.BlockSpec((_BLK, d), lambda i: (i, 0)),
        out_shape=jax.ShapeDtypeStruct((n_nodes, d), jnp.float32),
    )(self_vecs, neigh_vecs, w, b)

    return pl.pallas_call(
        _tc2_body,
        grid=(_NSC // _BLK2,),
        in_specs=[
            pl.BlockSpec((_BLK2, 2 * d), lambda i: (i, 0)),
            pl.BlockSpec((2 * d, d), lambda i: (0, 0)),
            pl.BlockSpec((1, d), lambda i: (0, 0)),
            pl.BlockSpec(memory_space=pl.ANY),
        ],
        out_specs=pl.BlockSpec((_BLK2, d), lambda i: (i + _NTC // _BLK2, 0)),
        out_shape=jax.ShapeDtypeStruct((n_nodes, d), jnp.float32),
        input_output_aliases={3: 0},
    )(sc_sums, w, b, out_head)


# speedup vs baseline: 2.6072x; 1.4195x over previous
import jax
import jax.numpy as jnp
from jax.experimental import pallas as pl

_BLK = 400


def _body(s_ref, n_ref, ws_ref, wn_ref, bs_ref, bn_ref, o_ref):
    inv_s = 1.0 / s_ref.shape[1]
    inv_n = 1.0 / n_ref.shape[1]
    smean = jnp.sum(s_ref[...], axis=1) * inv_s
    nmean = jnp.sum(n_ref[...], axis=1) * inv_n
    y = jnp.dot(smean, ws_ref[...], preferred_element_type=jnp.float32)
    y = y + jnp.dot(nmean, wn_ref[...], preferred_element_type=jnp.float32)
    o_ref[...] = jnp.maximum(y + (bs_ref[...] + bn_ref[...]), 0.0)


def kernel(self_vecs, neigh_vecs, W_neigh, b_neigh, W_self, b_self):
    n_nodes, s_self, d = self_vecs.shape
    s_neigh = neigh_vecs.shape[1]
    blk = _BLK
    grid = (n_nodes // blk,)

    return pl.pallas_call(
        _body,
        grid=grid,
        in_specs=[
            pl.BlockSpec((blk, s_self, d), lambda i: (i, 0, 0)),
            pl.BlockSpec((blk, s_neigh, d), lambda i: (i, 0, 0)),
            pl.BlockSpec((d, d), lambda i: (0, 0)),
            pl.BlockSpec((d, d), lambda i: (0, 0)),
            pl.BlockSpec((1, d), lambda i: (0, 0)),
            pl.BlockSpec((1, d), lambda i: (0, 0)),
        ],
        out_specs=pl.BlockSpec((blk, d), lambda i: (i, 0)),
        out_shape=jax.ShapeDtypeStruct((n_nodes, d), jnp.float32),
    )(self_vecs, neigh_vecs, W_self, W_neigh,
      b_self.reshape(1, d), b_neigh.reshape(1, d))
